# baseline (speedup 1.0000x reference)
import functools

import jax
import jax.numpy as jnp
from jax import lax
from jax.experimental import pallas as pl
from jax.experimental.pallas import tpu as pltpu
from jax.experimental.pallas import tpu_sc as plsc

E = 64
K = 2
D = 768
SW = 128
F = 2048
T = 2048
CAP = 160
DUMMY = E * CAP
XP_ROWS = (E + 1) * CAP
RB = 256
FBLK = 2048
FB = F // FBLK

NC, NS = 2, 16
NW = NC * NS
TPW = T // NW


def _router_body(x_ref, gw_ref, logits_ref, slot0_ref, slot1_ref,
                 ss0_ref, ss1_ref, counts_ref):
    x = x_ref[...]
    gw = gw_ref[...]
    logits = lax.dot_general(x, gw, (((1,), (1,)), ((), ())),
                             preferred_element_type=jnp.float32)
    logits_ref[...] = logits

    m = jnp.max(logits, axis=1, keepdims=True)
    p = jnp.exp(logits - m)
    probs = p / jnp.sum(p, axis=1, keepdims=True)

    lane = lax.broadcasted_iota(jnp.int32, (T, E), 1)
    p0 = jnp.max(probs, axis=1, keepdims=True)
    e0 = jnp.min(jnp.where(probs == p0, lane, E), axis=1, keepdims=True)
    probs1 = jnp.where(lane == e0, -1.0, probs)
    p1 = jnp.max(probs1, axis=1, keepdims=True)
    e1 = jnp.min(jnp.where(probs1 == p1, lane, E), axis=1, keepdims=True)
    den = p0 + p1
    s0 = p0 / den
    s1 = p1 / den

    tri = (lax.broadcasted_iota(jnp.int32, (RB, RB), 1)
           < lax.broadcasted_iota(jnp.int32, (RB, RB), 0)).astype(jnp.float32)
    lane_b = lax.broadcasted_iota(jnp.int32, (RB, E), 1)

    def prefix_pass(e_sel, run):
        parts = []
        for blk in range(T // RB):
            eb = lax.slice_in_dim(e_sel, blk * RB, (blk + 1) * RB, axis=0)
            oh = (lane_b == eb).astype(jnp.float32)
            excl = lax.dot_general(tri, oh, (((1,), (0,)), ((), ())),
                                   preferred_element_type=jnp.float32) + run
            parts.append(jnp.sum(excl * oh, axis=1, keepdims=True))
            run = run + jnp.sum(oh, axis=0, keepdims=True)
        return jnp.concatenate(parts, axis=0), run

    run0 = jnp.zeros((1, E), jnp.float32)
    rank0, run1 = prefix_pass(e0, run0)
    rank1, run2 = prefix_pass(e1, run1)
    counts_ref[...] = run2.astype(jnp.int32)

    def emit(e_sel, rank, s, slot_ref, ss_ref):
        r = rank.astype(jnp.int32)
        valid = r < CAP
        slot_ref[...] = jnp.where(valid, e_sel * CAP + r, DUMMY)
        ss_ref[...] = jnp.broadcast_to(jnp.where(valid, s, 0.0), (T, SW))

    emit(e0, rank0, s0, slot0_ref, ss0_ref)
    emit(e1, rank1, s1, slot1_ref, ss1_ref)


def _router_call(x, gate_w):
    return pl.pallas_call(
        _router_body,
        out_shape=(
            jax.ShapeDtypeStruct((T, E), jnp.float32),
            jax.ShapeDtypeStruct((T, 1), jnp.int32),
            jax.ShapeDtypeStruct((T, 1), jnp.int32),
            jax.ShapeDtypeStruct((T, SW), jnp.float32),
            jax.ShapeDtypeStruct((T, SW), jnp.float32),
            jax.ShapeDtypeStruct((1, E), jnp.int32),
        ),
    )(x, gate_w)


def _dispatch_body(x_hbm, ss0_hbm, ss1_hbm, slot0_hbm, slot1_hbm,
                   xp_hbm, ws_hbm,
                   idx0_v, idx1_v, rows_v, s0_v, s1_v, sem0, sem1):
    wid = lax.axis_index("s") * NC + lax.axis_index("c")
    base = wid * TPW
    pltpu.sync_copy(slot0_hbm.at[pl.ds(base, TPW)], idx0_v)
    pltpu.sync_copy(slot1_hbm.at[pl.ds(base, TPW)], idx1_v)
    pltpu.sync_copy(x_hbm.at[pl.ds(base, TPW)], rows_v)
    pltpu.sync_copy(ss0_hbm.at[pl.ds(base, TPW)], s0_v)
    pltpu.sync_copy(ss1_hbm.at[pl.ds(base, TPW)], s1_v)
    c0 = pltpu.async_copy(rows_v, xp_hbm.at[idx0_v], sem0)
    c1 = pltpu.async_copy(rows_v, xp_hbm.at[idx1_v], sem1)
    c2 = pltpu.async_copy(s0_v, ws_hbm.at[idx0_v], sem0)
    c3 = pltpu.async_copy(s1_v, ws_hbm.at[idx1_v], sem1)
    c0.wait()
    c1.wait()
    c2.wait()
    c3.wait()


@functools.cache
def _dispatch():
    return pl.kernel(
        _dispatch_body,
        out_type=(jax.ShapeDtypeStruct((XP_ROWS, D), jnp.float32),
                  jax.ShapeDtypeStruct((XP_ROWS, SW), jnp.float32)),
        mesh=plsc.VectorSubcoreMesh(core_axis_name="c", subcore_axis_name="s",
                                    num_cores=NC, num_subcores=NS),
        scratch_types=[
            pltpu.VMEM((TPW,), jnp.int32),
            pltpu.VMEM((TPW,), jnp.int32),
            pltpu.VMEM((TPW, D), jnp.float32),
            pltpu.VMEM((TPW, SW), jnp.float32),
            pltpu.VMEM((TPW, SW), jnp.float32),
            pltpu.SemaphoreType.DMA,
            pltpu.SemaphoreType.DMA,
        ],
    )


def _experts_body(counts_ref, xp_ref, ws_ref, w1_ref, w3_ref, w2_ref, yp_ref):
    xt = xp_ref[...]
    a = lax.dot_general(xt, w1_ref[0], (((1,), (1,)), ((), ())),
                        preferred_element_type=jnp.float32)
    b = lax.dot_general(xt, w3_ref[0], (((1,), (1,)), ((), ())),
                        preferred_element_type=jnp.float32)
    h = (a * (1.0 / (1.0 + jnp.exp(-a)))) * b
    contrib = lax.dot_general(h, w2_ref[0], (((1,), (1,)), ((), ())),
                              preferred_element_type=jnp.float32)
    e = pl.program_id(0)
    cnt = jnp.where(e < E, counts_ref[0, jnp.minimum(e, E - 1)], 0)
    rows = lax.broadcasted_iota(jnp.int32, (CAP, D), 0)
    scale = ws_ref[:, 0:1]
    yp_ref[...] = jnp.where(rows < cnt, contrib * scale, 0.0)


def _experts_call(counts, xp, ws, w1, w3, w2):
    ec = lambda e: jnp.minimum(e, E - 1)
    return pl.pallas_call(
        _experts_body,
        grid=(E + 1,),
        in_specs=[
            pl.BlockSpec(memory_space=pltpu.SMEM),
            pl.BlockSpec((CAP, D), lambda e: (e, 0)),
            pl.BlockSpec((CAP, SW), lambda e: (e, 0)),
            pl.BlockSpec((1, F, D), lambda e: (ec(e), 0, 0)),
            pl.BlockSpec((1, F, D), lambda e: (ec(e), 0, 0)),
            pl.BlockSpec((1, D, F), lambda e: (ec(e), 0, 0)),
        ],
        out_specs=pl.BlockSpec((CAP, D), lambda e: (e, 0)),
        out_shape=jax.ShapeDtypeStruct((XP_ROWS, D), jnp.float32),
    )(counts, xp, ws, w1, w3, w2)


def _combine_body(yp_hbm, slot0_hbm, slot1_hbm, out_hbm,
                  idx0_v, idx1_v, bufa, bufb, sem0, sem1):
    wid = lax.axis_index("s") * NC + lax.axis_index("c")
    base = wid * TPW
    pltpu.sync_copy(slot0_hbm.at[pl.ds(base, TPW)], idx0_v)
    pltpu.sync_copy(slot1_hbm.at[pl.ds(base, TPW)], idx1_v)
    ca = pltpu.async_copy(yp_hbm.at[idx0_v], bufa, sem0)
    cb = pltpu.async_copy(yp_hbm.at[idx1_v], bufb, sem1)
    ca.wait()
    cb.wait()

    def tok_body(t, carry):
        for j in range(D // 16):
            sl = pl.ds(j * 16, 16)
            bufa[t, sl] = bufa[t, sl] + bufb[t, sl]
        return carry

    lax.fori_loop(0, TPW, tok_body, 0)
    pltpu.sync_copy(bufa, out_hbm.at[pl.ds(base, TPW)])


@functools.cache
def _combine():
    return pl.kernel(
        _combine_body,
        out_type=jax.ShapeDtypeStruct((T, D), jnp.float32),
        mesh=plsc.VectorSubcoreMesh(core_axis_name="c", subcore_axis_name="s",
                                    num_cores=NC, num_subcores=NS),
        scratch_types=[
            pltpu.VMEM((TPW,), jnp.int32),
            pltpu.VMEM((TPW,), jnp.int32),
            pltpu.VMEM((TPW, D), jnp.float32),
            pltpu.VMEM((TPW, D), jnp.float32),
            pltpu.SemaphoreType.DMA,
            pltpu.SemaphoreType.DMA,
        ],
    )


def kernel(hidden_states, gate_w, w1, w3, w2):
    b, s, d = hidden_states.shape
    x = hidden_states.reshape(b * s, d)
    logits, slot0, slot1, ss0, ss1, counts = _router_call(x, gate_w)
    slot0 = slot0.reshape(T)
    slot1 = slot1.reshape(T)
    xp, ws = _dispatch()(x, ss0, ss1, slot0, slot1)
    yp = _experts_call(counts, xp, ws, w1, w3, w2)
    out = _combine()(yp, slot0, slot1)
    return out.reshape(b, s, d), logits


# speedup vs baseline: 1.0042x; 1.0042x over previous
import functools

import jax
import jax.numpy as jnp
from jax import lax
from jax.experimental import pallas as pl
from jax.experimental.pallas import tpu as pltpu
from jax.experimental.pallas import tpu_sc as plsc

E = 64
K = 2
D = 768
SW = 128
F = 2048
T = 2048
CAP = 160
DUMMY = E * CAP
XP_ROWS = (E + 1) * CAP
RB = 256
FBLK = 2048
FB = F // FBLK

NC, NS = 2, 16
NW = NC * NS
TPW = T // NW


def _router_body(x_ref, gw_ref, logits_ref, slot0_ref, slot1_ref,
                 ss0_ref, ss1_ref, counts_ref):
    x = x_ref[...]
    gw = gw_ref[...]
    logits = lax.dot_general(x, gw, (((1,), (1,)), ((), ())),
                             preferred_element_type=jnp.float32)
    logits_ref[...] = logits

    m = jnp.max(logits, axis=1, keepdims=True)
    p = jnp.exp(logits - m)
    probs = p / jnp.sum(p, axis=1, keepdims=True)

    lane = lax.broadcasted_iota(jnp.int32, (T, E), 1)
    p0 = jnp.max(probs, axis=1, keepdims=True)
    e0 = jnp.min(jnp.where(probs == p0, lane, E), axis=1, keepdims=True)
    probs1 = jnp.where(lane == e0, -1.0, probs)
    p1 = jnp.max(probs1, axis=1, keepdims=True)
    e1 = jnp.min(jnp.where(probs1 == p1, lane, E), axis=1, keepdims=True)
    den = p0 + p1
    s0 = p0 / den
    s1 = p1 / den

    tri = (lax.broadcasted_iota(jnp.int32, (RB, RB), 1)
           < lax.broadcasted_iota(jnp.int32, (RB, RB), 0)).astype(jnp.float32)
    lane_b = lax.broadcasted_iota(jnp.int32, (RB, E), 1)

    def prefix_pass(e_sel, run):
        parts = []
        for blk in range(T // RB):
            eb = lax.slice_in_dim(e_sel, blk * RB, (blk + 1) * RB, axis=0)
            oh = (lane_b == eb).astype(jnp.float32)
            excl = lax.dot_general(tri, oh, (((1,), (0,)), ((), ())),
                                   preferred_element_type=jnp.float32) + run
            parts.append(jnp.sum(excl * oh, axis=1, keepdims=True))
            run = run + jnp.sum(oh, axis=0, keepdims=True)
        return jnp.concatenate(parts, axis=0), run

    run0 = jnp.zeros((1, E), jnp.float32)
    rank0, run1 = prefix_pass(e0, run0)
    rank1, run2 = prefix_pass(e1, run1)
    counts_ref[...] = run2.astype(jnp.int32)

    def emit(e_sel, rank, s, slot_ref, ss_ref):
        r = rank.astype(jnp.int32)
        valid = r < CAP
        slot_ref[...] = jnp.where(valid, e_sel * CAP + r, DUMMY)
        ss_ref[...] = jnp.broadcast_to(jnp.where(valid, s, 0.0), (T, SW))

    emit(e0, rank0, s0, slot0_ref, ss0_ref)
    emit(e1, rank1, s1, slot1_ref, ss1_ref)


def _router_call(x, gate_w):
    return pl.pallas_call(
        _router_body,
        out_shape=(
            jax.ShapeDtypeStruct((T, E), jnp.float32),
            jax.ShapeDtypeStruct((T, 1), jnp.int32),
            jax.ShapeDtypeStruct((T, 1), jnp.int32),
            jax.ShapeDtypeStruct((T, SW), jnp.float32),
            jax.ShapeDtypeStruct((T, SW), jnp.float32),
            jax.ShapeDtypeStruct((1, E), jnp.int32),
        ),
    )(x, gate_w)


def _dispatch_body(x_hbm, slot0_hbm, slot1_hbm, xp_hbm,
                   idx0_v, idx1_v, rows_v, sem0, sem1):
    wid = lax.axis_index("s") * NC + lax.axis_index("c")
    base = wid * TPW
    pltpu.sync_copy(slot0_hbm.at[pl.ds(base, TPW)], idx0_v)
    pltpu.sync_copy(slot1_hbm.at[pl.ds(base, TPW)], idx1_v)
    pltpu.sync_copy(x_hbm.at[pl.ds(base, TPW)], rows_v)
    c0 = pltpu.async_copy(rows_v, xp_hbm.at[idx0_v], sem0)
    c1 = pltpu.async_copy(rows_v, xp_hbm.at[idx1_v], sem1)
    c0.wait()
    c1.wait()


@functools.cache
def _dispatch():
    return pl.kernel(
        _dispatch_body,
        out_type=jax.ShapeDtypeStruct((XP_ROWS, D), jnp.float32),
        mesh=plsc.VectorSubcoreMesh(core_axis_name="c", subcore_axis_name="s",
                                    num_cores=NC, num_subcores=NS),
        scratch_types=[
            pltpu.VMEM((TPW,), jnp.int32),
            pltpu.VMEM((TPW,), jnp.int32),
            pltpu.VMEM((TPW, D), jnp.float32),
            pltpu.SemaphoreType.DMA,
            pltpu.SemaphoreType.DMA,
        ],
    )


def _experts_body(counts_ref, xp_ref, w1_ref, w3_ref, w2_ref, yp_ref):
    xt = xp_ref[...]
    a = lax.dot_general(xt, w1_ref[0], (((1,), (1,)), ((), ())),
                        preferred_element_type=jnp.float32)
    b = lax.dot_general(xt, w3_ref[0], (((1,), (1,)), ((), ())),
                        preferred_element_type=jnp.float32)
    h = (a * (1.0 / (1.0 + jnp.exp(-a)))) * b
    contrib = lax.dot_general(h, w2_ref[0], (((1,), (1,)), ((), ())),
                              preferred_element_type=jnp.float32)
    e = pl.program_id(0)
    cnt = jnp.where(e < E, counts_ref[0, jnp.minimum(e, E - 1)], 0)
    rows = lax.broadcasted_iota(jnp.int32, (CAP, D), 0)
    yp_ref[...] = jnp.where(rows < cnt, contrib, 0.0)


def _experts_call(counts, xp, w1, w3, w2):
    ec = lambda e: jnp.minimum(e, E - 1)
    return pl.pallas_call(
        _experts_body,
        grid=(E + 1,),
        in_specs=[
            pl.BlockSpec(memory_space=pltpu.SMEM),
            pl.BlockSpec((CAP, D), lambda e: (e, 0)),
            pl.BlockSpec((1, F, D), lambda e: (ec(e), 0, 0)),
            pl.BlockSpec((1, F, D), lambda e: (ec(e), 0, 0)),
            pl.BlockSpec((1, D, F), lambda e: (ec(e), 0, 0)),
        ],
        out_specs=pl.BlockSpec((CAP, D), lambda e: (e, 0)),
        out_shape=jax.ShapeDtypeStruct((XP_ROWS, D), jnp.float32),
    )(counts, xp, w1, w3, w2)


HTPW = TPW // 2


def _combine_body(yp_hbm, ss0_hbm, ss1_hbm, slot0_hbm, slot1_hbm, out_hbm,
                  idx00_v, idx01_v, idx10_v, idx11_v, ss0_v, ss1_v,
                  bufa0, bufb0, bufa1, bufb1,
                  semA0, semB0, semA1, semB1, semO0, semO1):
    wid = lax.axis_index("s") * NC + lax.axis_index("c")
    base = wid * TPW
    pltpu.sync_copy(slot0_hbm.at[pl.ds(base, HTPW)], idx00_v)
    pltpu.sync_copy(slot1_hbm.at[pl.ds(base, HTPW)], idx10_v)
    pltpu.sync_copy(slot0_hbm.at[pl.ds(base + HTPW, HTPW)], idx01_v)
    pltpu.sync_copy(slot1_hbm.at[pl.ds(base + HTPW, HTPW)], idx11_v)
    g0a = pltpu.async_copy(yp_hbm.at[idx00_v], bufa0, semA0)
    g0b = pltpu.async_copy(yp_hbm.at[idx10_v], bufb0, semB0)
    g1a = pltpu.async_copy(yp_hbm.at[idx01_v], bufa1, semA1)
    g1b = pltpu.async_copy(yp_hbm.at[idx11_v], bufb1, semB1)
    pltpu.sync_copy(ss0_hbm.at[pl.ds(base, TPW)], ss0_v)
    pltpu.sync_copy(ss1_hbm.at[pl.ds(base, TPW)], ss1_v)

    def make_tok_body(ba, bb, off):
        def tok_body(t, carry):
            s0 = ss0_v[off + t, pl.ds(0, 16)]
            s1 = ss1_v[off + t, pl.ds(0, 16)]
            for j in range(D // 16):
                sl = pl.ds(j * 16, 16)
                ba[t, sl] = ba[t, sl] * s0 + bb[t, sl] * s1
            return carry
        return tok_body

    g0a.wait()
    g0b.wait()
    lax.fori_loop(0, HTPW, make_tok_body(bufa0, bufb0, 0), 0)
    o0 = pltpu.async_copy(bufa0, out_hbm.at[pl.ds(base, HTPW)], semO0)
    g1a.wait()
    g1b.wait()
    lax.fori_loop(0, HTPW, make_tok_body(bufa1, bufb1, HTPW), 0)
    o1 = pltpu.async_copy(bufa1, out_hbm.at[pl.ds(base + HTPW, HTPW)], semO1)
    o0.wait()
    o1.wait()


@functools.cache
def _combine():
    return pl.kernel(
        _combine_body,
        out_type=jax.ShapeDtypeStruct((T, D), jnp.float32),
        mesh=plsc.VectorSubcoreMesh(core_axis_name="c", subcore_axis_name="s",
                                    num_cores=NC, num_subcores=NS),
        scratch_types=[
            pltpu.VMEM((HTPW,), jnp.int32),
            pltpu.VMEM((HTPW,), jnp.int32),
            pltpu.VMEM((HTPW,), jnp.int32),
            pltpu.VMEM((HTPW,), jnp.int32),
            pltpu.VMEM((TPW, SW), jnp.float32),
            pltpu.VMEM((TPW, SW), jnp.float32),
            pltpu.VMEM((HTPW, D), jnp.float32),
            pltpu.VMEM((HTPW, D), jnp.float32),
            pltpu.VMEM((HTPW, D), jnp.float32),
            pltpu.VMEM((HTPW, D), jnp.float32),
            pltpu.SemaphoreType.DMA,
            pltpu.SemaphoreType.DMA,
            pltpu.SemaphoreType.DMA,
            pltpu.SemaphoreType.DMA,
            pltpu.SemaphoreType.DMA,
            pltpu.SemaphoreType.DMA,
        ],
    )


def kernel(hidden_states, gate_w, w1, w3, w2):
    b, s, d = hidden_states.shape
    x = hidden_states.reshape(b * s, d)
    logits, slot0, slot1, ss0, ss1, counts = _router_call(x, gate_w)
    slot0 = slot0.reshape(T)
    slot1 = slot1.reshape(T)
    xp = _dispatch()(x, slot0, slot1)
    yp = _experts_call(counts, xp, w1, w3, w2)
    out = _combine()(yp, ss0, ss1, slot0, slot1)
    return out.reshape(b, s, d), logits


# speedup vs baseline: 1.0057x; 1.0015x over previous
import functools

import jax
import jax.numpy as jnp
from jax import lax
from jax.experimental import pallas as pl
from jax.experimental.pallas import tpu as pltpu
from jax.experimental.pallas import tpu_sc as plsc

E = 64
K = 2
D = 768
SW = 128
F = 2048
T = 2048
CAP = 160
DUMMY = E * CAP
XP_ROWS = (E + 1) * CAP
RB = 256
FBLK = 2048
FB = F // FBLK

NC, NS = 2, 16
NW = NC * NS
TPW = T // NW


def _router_body(x_ref, gw_ref, logits_ref, slot0_ref, slot1_ref,
                 ss0_ref, ss1_ref, counts_ref):
    x = x_ref[...]
    gw = gw_ref[...]
    logits = lax.dot_general(x, gw, (((1,), (1,)), ((), ())),
                             preferred_element_type=jnp.float32)
    logits_ref[...] = logits

    m = jnp.max(logits, axis=1, keepdims=True)
    p = jnp.exp(logits - m)
    probs = p / jnp.sum(p, axis=1, keepdims=True)

    lane = lax.broadcasted_iota(jnp.int32, (T, E), 1)
    p0 = jnp.max(probs, axis=1, keepdims=True)
    e0 = jnp.min(jnp.where(probs == p0, lane, E), axis=1, keepdims=True)
    probs1 = jnp.where(lane == e0, -1.0, probs)
    p1 = jnp.max(probs1, axis=1, keepdims=True)
    e1 = jnp.min(jnp.where(probs1 == p1, lane, E), axis=1, keepdims=True)
    den = p0 + p1
    s0 = p0 / den
    s1 = p1 / den

    tri = (lax.broadcasted_iota(jnp.int32, (RB, RB), 1)
           < lax.broadcasted_iota(jnp.int32, (RB, RB), 0)).astype(jnp.float32)
    lane_b = lax.broadcasted_iota(jnp.int32, (RB, E), 1)

    def prefix_pass(e_sel, run):
        parts = []
        for blk in range(T // RB):
            eb = lax.slice_in_dim(e_sel, blk * RB, (blk + 1) * RB, axis=0)
            oh = (lane_b == eb).astype(jnp.float32)
            excl = lax.dot_general(tri, oh, (((1,), (0,)), ((), ())),
                                   preferred_element_type=jnp.float32) + run
            parts.append(jnp.sum(excl * oh, axis=1, keepdims=True))
            run = run + jnp.sum(oh, axis=0, keepdims=True)
        return jnp.concatenate(parts, axis=0), run

    run0 = jnp.zeros((1, E), jnp.float32)
    rank0, run1 = prefix_pass(e0, run0)
    rank1, run2 = prefix_pass(e1, run1)
    counts_ref[...] = run2.astype(jnp.int32)

    def emit(e_sel, rank, s, slot_ref, ss_ref):
        r = rank.astype(jnp.int32)
        valid = r < CAP
        slot_ref[...] = jnp.where(valid, e_sel * CAP + r, DUMMY)
        ss_ref[...] = jnp.broadcast_to(jnp.where(valid, s, 0.0), (T, SW))

    emit(e0, rank0, s0, slot0_ref, ss0_ref)
    emit(e1, rank1, s1, slot1_ref, ss1_ref)


def _router_call(x, gate_w):
    return pl.pallas_call(
        _router_body,
        out_shape=(
            jax.ShapeDtypeStruct((T, E), jnp.float32),
            jax.ShapeDtypeStruct((T, 1), jnp.int32),
            jax.ShapeDtypeStruct((T, 1), jnp.int32),
            jax.ShapeDtypeStruct((T, SW), jnp.float32),
            jax.ShapeDtypeStruct((T, SW), jnp.float32),
            jax.ShapeDtypeStruct((1, E), jnp.int32),
        ),
    )(x, gate_w)


def _dispatch_body(x_hbm, slot0_hbm, slot1_hbm, xp_hbm,
                   idx0_v, idx1_v, rows_v, sem0, sem1, sem2):
    wid = lax.axis_index("s") * NC + lax.axis_index("c")
    base = wid * TPW
    l0 = pltpu.async_copy(slot0_hbm.at[pl.ds(base, TPW)], idx0_v, sem0)
    l1 = pltpu.async_copy(slot1_hbm.at[pl.ds(base, TPW)], idx1_v, sem1)
    l2 = pltpu.async_copy(x_hbm.at[pl.ds(base, TPW)], rows_v, sem2)
    l0.wait()
    l1.wait()
    l2.wait()
    c0 = pltpu.async_copy(rows_v, xp_hbm.at[idx0_v], sem0)
    c1 = pltpu.async_copy(rows_v, xp_hbm.at[idx1_v], sem1)
    c0.wait()
    c1.wait()


@functools.cache
def _dispatch():
    return pl.kernel(
        _dispatch_body,
        out_type=jax.ShapeDtypeStruct((XP_ROWS, D), jnp.float32),
        mesh=plsc.VectorSubcoreMesh(core_axis_name="c", subcore_axis_name="s",
                                    num_cores=NC, num_subcores=NS),
        scratch_types=[
            pltpu.VMEM((TPW,), jnp.int32),
            pltpu.VMEM((TPW,), jnp.int32),
            pltpu.VMEM((TPW, D), jnp.float32),
            pltpu.SemaphoreType.DMA,
            pltpu.SemaphoreType.DMA,
            pltpu.SemaphoreType.DMA,
        ],
    )


def _experts_body(counts_ref, xp_ref, w1_ref, w3_ref, w2_ref, yp_ref):
    xt = xp_ref[...]
    a = lax.dot_general(xt, w1_ref[0], (((1,), (1,)), ((), ())),
                        preferred_element_type=jnp.float32)
    b = lax.dot_general(xt, w3_ref[0], (((1,), (1,)), ((), ())),
                        preferred_element_type=jnp.float32)
    h = (a * (1.0 / (1.0 + jnp.exp(-a)))) * b
    contrib = lax.dot_general(h, w2_ref[0], (((1,), (1,)), ((), ())),
                              preferred_element_type=jnp.float32)
    e = pl.program_id(0)
    cnt = jnp.where(e < E, counts_ref[0, jnp.minimum(e, E - 1)], 0)
    rows = lax.broadcasted_iota(jnp.int32, (CAP, D), 0)
    yp_ref[...] = jnp.where(rows < cnt, contrib, 0.0)


def _experts_call(counts, xp, w1, w3, w2):
    ec = lambda e: jnp.minimum(e, E - 1)
    return pl.pallas_call(
        _experts_body,
        grid=(E + 1,),
        in_specs=[
            pl.BlockSpec(memory_space=pltpu.SMEM),
            pl.BlockSpec((CAP, D), lambda e: (e, 0)),
            pl.BlockSpec((1, F, D), lambda e: (ec(e), 0, 0)),
            pl.BlockSpec((1, F, D), lambda e: (ec(e), 0, 0)),
            pl.BlockSpec((1, D, F), lambda e: (ec(e), 0, 0)),
        ],
        out_specs=pl.BlockSpec((CAP, D), lambda e: (e, 0)),
        out_shape=jax.ShapeDtypeStruct((XP_ROWS, D), jnp.float32),
    )(counts, xp, w1, w3, w2)


HTPW = TPW // 2


def _combine_body(yp_hbm, ss0_hbm, ss1_hbm, slot0_hbm, slot1_hbm, out_hbm,
                  idx00_v, idx01_v, idx10_v, idx11_v, ss0_v, ss1_v,
                  bufa0, bufb0, bufa1, bufb1,
                  semA0, semB0, semA1, semB1, semO0, semO1):
    wid = lax.axis_index("s") * NC + lax.axis_index("c")
    base = wid * TPW
    pltpu.sync_copy(slot0_hbm.at[pl.ds(base, HTPW)], idx00_v)
    pltpu.sync_copy(slot1_hbm.at[pl.ds(base, HTPW)], idx10_v)
    pltpu.sync_copy(slot0_hbm.at[pl.ds(base + HTPW, HTPW)], idx01_v)
    pltpu.sync_copy(slot1_hbm.at[pl.ds(base + HTPW, HTPW)], idx11_v)
    g0a = pltpu.async_copy(yp_hbm.at[idx00_v], bufa0, semA0)
    g0b = pltpu.async_copy(yp_hbm.at[idx10_v], bufb0, semB0)
    g1a = pltpu.async_copy(yp_hbm.at[idx01_v], bufa1, semA1)
    g1b = pltpu.async_copy(yp_hbm.at[idx11_v], bufb1, semB1)
    pltpu.sync_copy(ss0_hbm.at[pl.ds(base, TPW)], ss0_v)
    pltpu.sync_copy(ss1_hbm.at[pl.ds(base, TPW)], ss1_v)

    def make_tok_body(ba, bb, off):
        def tok_body(t, carry):
            s0 = ss0_v[off + t, pl.ds(0, 16)]
            s1 = ss1_v[off + t, pl.ds(0, 16)]
            for j in range(D // 16):
                sl = pl.ds(j * 16, 16)
                ba[t, sl] = ba[t, sl] * s0 + bb[t, sl] * s1
            return carry
        return tok_body

    g0a.wait()
    g0b.wait()
    lax.fori_loop(0, HTPW, make_tok_body(bufa0, bufb0, 0), 0)
    o0 = pltpu.async_copy(bufa0, out_hbm.at[pl.ds(base, HTPW)], semO0)
    g1a.wait()
    g1b.wait()
    lax.fori_loop(0, HTPW, make_tok_body(bufa1, bufb1, HTPW), 0)
    o1 = pltpu.async_copy(bufa1, out_hbm.at[pl.ds(base + HTPW, HTPW)], semO1)
    o0.wait()
    o1.wait()


@functools.cache
def _combine():
    return pl.kernel(
        _combine_body,
        out_type=jax.ShapeDtypeStruct((T, D), jnp.float32),
        mesh=plsc.VectorSubcoreMesh(core_axis_name="c", subcore_axis_name="s",
                                    num_cores=NC, num_subcores=NS),
        scratch_types=[
            pltpu.VMEM((HTPW,), jnp.int32),
            pltpu.VMEM((HTPW,), jnp.int32),
            pltpu.VMEM((HTPW,), jnp.int32),
            pltpu.VMEM((HTPW,), jnp.int32),
            pltpu.VMEM((TPW, SW), jnp.float32),
            pltpu.VMEM((TPW, SW), jnp.float32),
            pltpu.VMEM((HTPW, D), jnp.float32),
            pltpu.VMEM((HTPW, D), jnp.float32),
            pltpu.VMEM((HTPW, D), jnp.float32),
            pltpu.VMEM((HTPW, D), jnp.float32),
            pltpu.SemaphoreType.DMA,
            pltpu.SemaphoreType.DMA,
            pltpu.SemaphoreType.DMA,
            pltpu.SemaphoreType.DMA,
            pltpu.SemaphoreType.DMA,
            pltpu.SemaphoreType.DMA,
        ],
    )


def kernel(hidden_states, gate_w, w1, w3, w2):
    b, s, d = hidden_states.shape
    x = hidden_states.reshape(b * s, d)
    logits, slot0, slot1, ss0, ss1, counts = _router_call(x, gate_w)
    slot0 = slot0.reshape(T)
    slot1 = slot1.reshape(T)
    xp = _dispatch()(x, slot0, slot1)
    yp = _experts_call(counts, xp, w1, w3, w2)
    out = _combine()(yp, ss0, ss1, slot0, slot1)
    return out.reshape(b, s, d), logits


# speedup vs baseline: 1.0063x; 1.0006x over previous
import functools

import jax
import jax.numpy as jnp
from jax import lax
from jax.experimental import pallas as pl
from jax.experimental.pallas import tpu as pltpu
from jax.experimental.pallas import tpu_sc as plsc

E = 64
K = 2
D = 768
SW = 128
F = 2048
T = 2048
CAP = 160
DUMMY = E * CAP
XP_ROWS = (E + 1) * CAP
RB = 256

NC, NS = 2, 16
NW = NC * NS
TPW = T // NW


def _router_body(x_ref, gw_ref, logits_ref, slot0_ref, slot1_ref,
                 ss0_ref, ss1_ref, counts_ref):
    x = x_ref[...]
    gw = gw_ref[...]
    logits = lax.dot_general(x, gw, (((1,), (1,)), ((), ())),
                             preferred_element_type=jnp.float32)
    logits_ref[...] = logits

    m = jnp.max(logits, axis=1, keepdims=True)
    p = jnp.exp(logits - m)
    probs = p / jnp.sum(p, axis=1, keepdims=True)

    lane = lax.broadcasted_iota(jnp.int32, (T, E), 1)
    p0 = jnp.max(probs, axis=1, keepdims=True)
    e0 = jnp.min(jnp.where(probs == p0, lane, E), axis=1, keepdims=True)
    probs1 = jnp.where(lane == e0, -1.0, probs)
    p1 = jnp.max(probs1, axis=1, keepdims=True)
    e1 = jnp.min(jnp.where(probs1 == p1, lane, E), axis=1, keepdims=True)
    den = p0 + p1
    s0 = p0 / den
    s1 = p1 / den

    tri = (lax.broadcasted_iota(jnp.int32, (RB, RB), 1)
           < lax.broadcasted_iota(jnp.int32, (RB, RB), 0)).astype(jnp.float32)
    lane_b = lax.broadcasted_iota(jnp.int32, (RB, E), 1)

    def prefix_pass(e_sel, run):
        parts = []
        for blk in range(T // RB):
            eb = lax.slice_in_dim(e_sel, blk * RB, (blk + 1) * RB, axis=0)
            oh = (lane_b == eb).astype(jnp.float32)
            excl = lax.dot_general(tri, oh, (((1,), (0,)), ((), ())),
                                   preferred_element_type=jnp.float32) + run
            parts.append(jnp.sum(excl * oh, axis=1, keepdims=True))
            run = run + jnp.sum(oh, axis=0, keepdims=True)
        return jnp.concatenate(parts, axis=0), run

    run0 = jnp.zeros((1, E), jnp.float32)
    rank0, run1 = prefix_pass(e0, run0)
    rank1, run2 = prefix_pass(e1, run1)
    counts_ref[...] = run2.astype(jnp.int32)

    def emit(e_sel, rank, s, slot_ref, ss_ref):
        r = rank.astype(jnp.int32)
        valid = r < CAP
        slot_ref[...] = jnp.where(valid, e_sel * CAP + r, DUMMY)
        ss_ref[...] = jnp.broadcast_to(jnp.where(valid, s, 0.0), (T, SW))

    emit(e0, rank0, s0, slot0_ref, ss0_ref)
    emit(e1, rank1, s1, slot1_ref, ss1_ref)


def _router_call(x, gate_w):
    return pl.pallas_call(
        _router_body,
        out_shape=(
            jax.ShapeDtypeStruct((T, E), jnp.float32),
            jax.ShapeDtypeStruct((T, 1), jnp.int32),
            jax.ShapeDtypeStruct((T, 1), jnp.int32),
            jax.ShapeDtypeStruct((T, SW), jnp.float32),
            jax.ShapeDtypeStruct((T, SW), jnp.float32),
            jax.ShapeDtypeStruct((1, E), jnp.int32),
        ),
    )(x, gate_w)


def _dispatch_body(x_hbm, slot0_hbm, slot1_hbm, xp_hbm,
                   idx0_v, idx1_v, rows_v, sem0, sem1, sem2):
    wid = lax.axis_index("s") * NC + lax.axis_index("c")
    base = wid * TPW
    l0 = pltpu.async_copy(slot0_hbm.at[pl.ds(base, TPW)], idx0_v, sem0)
    l1 = pltpu.async_copy(slot1_hbm.at[pl.ds(base, TPW)], idx1_v, sem1)
    l2 = pltpu.async_copy(x_hbm.at[pl.ds(base, TPW)], rows_v, sem2)
    l0.wait()
    l1.wait()
    l2.wait()
    c0 = pltpu.async_copy(rows_v, xp_hbm.at[idx0_v], sem0)
    c1 = pltpu.async_copy(rows_v, xp_hbm.at[idx1_v], sem1)
    c0.wait()
    c1.wait()


@functools.cache
def _dispatch():
    return pl.kernel(
        _dispatch_body,
        out_type=jax.ShapeDtypeStruct((XP_ROWS, D), jnp.float32),
        mesh=plsc.VectorSubcoreMesh(core_axis_name="c", subcore_axis_name="s",
                                    num_cores=NC, num_subcores=NS),
        scratch_types=[
            pltpu.VMEM((TPW,), jnp.int32),
            pltpu.VMEM((TPW,), jnp.int32),
            pltpu.VMEM((TPW, D), jnp.float32),
            pltpu.SemaphoreType.DMA,
            pltpu.SemaphoreType.DMA,
            pltpu.SemaphoreType.DMA,
        ],
    )


def _experts_body(counts_ref, xp_ref, w1_ref, w3_ref, w2_ref, yp_ref):
    xt = xp_ref[...]
    a = lax.dot_general(xt, w1_ref[0], (((1,), (1,)), ((), ())),
                        preferred_element_type=jnp.float32)
    b = lax.dot_general(xt, w3_ref[0], (((1,), (1,)), ((), ())),
                        preferred_element_type=jnp.float32)
    h = (a * (1.0 / (1.0 + jnp.exp(-a)))) * b
    contrib = lax.dot_general(h, w2_ref[0], (((1,), (1,)), ((), ())),
                              preferred_element_type=jnp.float32)
    e = pl.program_id(0)
    cnt = jnp.where(e < E, counts_ref[0, jnp.minimum(e, E - 1)], 0)
    rows = lax.broadcasted_iota(jnp.int32, (CAP, D), 0)
    yp_ref[...] = jnp.where(rows < cnt, contrib, 0.0)


def _experts_call(counts, xp, w1, w3, w2):
    ec = lambda e: jnp.minimum(e, E - 1)
    return pl.pallas_call(
        _experts_body,
        grid=(E + 1,),
        in_specs=[
            pl.BlockSpec(memory_space=pltpu.SMEM),
            pl.BlockSpec((CAP, D), lambda e: (e, 0)),
            pl.BlockSpec((1, F, D), lambda e: (ec(e), 0, 0)),
            pl.BlockSpec((1, F, D), lambda e: (ec(e), 0, 0)),
            pl.BlockSpec((1, D, F), lambda e: (ec(e), 0, 0)),
        ],
        out_specs=pl.BlockSpec((CAP, D), lambda e: (e, 0)),
        out_shape=jax.ShapeDtypeStruct((XP_ROWS, D), jnp.float32),
    )(counts, xp, w1, w3, w2)


HTPW = TPW // 2


def _combine_body(yp_hbm, ss0_hbm, ss1_hbm, slot0_hbm, slot1_hbm, out_hbm,
                  idx00_v, idx01_v, idx10_v, idx11_v, ss0_v, ss1_v,
                  bufa0, bufb0, bufa1, bufb1,
                  semA0, semB0, semA1, semB1, semO0, semO1):
    wid = lax.axis_index("s") * NC + lax.axis_index("c")
    base = wid * TPW
    pltpu.sync_copy(slot0_hbm.at[pl.ds(base, HTPW)], idx00_v)
    pltpu.sync_copy(slot1_hbm.at[pl.ds(base, HTPW)], idx10_v)
    pltpu.sync_copy(slot0_hbm.at[pl.ds(base + HTPW, HTPW)], idx01_v)
    pltpu.sync_copy(slot1_hbm.at[pl.ds(base + HTPW, HTPW)], idx11_v)
    g0a = pltpu.async_copy(yp_hbm.at[idx00_v], bufa0, semA0)
    g0b = pltpu.async_copy(yp_hbm.at[idx10_v], bufb0, semB0)
    g1a = pltpu.async_copy(yp_hbm.at[idx01_v], bufa1, semA1)
    g1b = pltpu.async_copy(yp_hbm.at[idx11_v], bufb1, semB1)
    pltpu.sync_copy(ss0_hbm.at[pl.ds(base, TPW)], ss0_v)
    pltpu.sync_copy(ss1_hbm.at[pl.ds(base, TPW)], ss1_v)

    def make_tok_body(ba, bb, off):
        def tok_body(t, carry):
            s0 = ss0_v[off + t, pl.ds(0, 16)]
            s1 = ss1_v[off + t, pl.ds(0, 16)]
            for j in range(D // 16):
                sl = pl.ds(j * 16, 16)
                ba[t, sl] = ba[t, sl] * s0 + bb[t, sl] * s1
            return carry
        return tok_body

    g0a.wait()
    g0b.wait()
    lax.fori_loop(0, HTPW, make_tok_body(bufa0, bufb0, 0), 0)
    o0 = pltpu.async_copy(bufa0, out_hbm.at[pl.ds(base, HTPW)], semO0)
    g1a.wait()
    g1b.wait()
    lax.fori_loop(0, HTPW, make_tok_body(bufa1, bufb1, HTPW), 0)
    o1 = pltpu.async_copy(bufa1, out_hbm.at[pl.ds(base + HTPW, HTPW)], semO1)
    o0.wait()
    o1.wait()


@functools.cache
def _combine():
    return pl.kernel(
        _combine_body,
        out_type=jax.ShapeDtypeStruct((T, D), jnp.float32),
        mesh=plsc.VectorSubcoreMesh(core_axis_name="c", subcore_axis_name="s",
                                    num_cores=NC, num_subcores=NS),
        scratch_types=[
            pltpu.VMEM((HTPW,), jnp.int32),
            pltpu.VMEM((HTPW,), jnp.int32),
            pltpu.VMEM((HTPW,), jnp.int32),
            pltpu.VMEM((HTPW,), jnp.int32),
            pltpu.VMEM((TPW, SW), jnp.float32),
            pltpu.VMEM((TPW, SW), jnp.float32),
            pltpu.VMEM((HTPW, D), jnp.float32),
            pltpu.VMEM((HTPW, D), jnp.float32),
            pltpu.VMEM((HTPW, D), jnp.float32),
            pltpu.VMEM((HTPW, D), jnp.float32),
            pltpu.SemaphoreType.DMA,
            pltpu.SemaphoreType.DMA,
            pltpu.SemaphoreType.DMA,
            pltpu.SemaphoreType.DMA,
            pltpu.SemaphoreType.DMA,
            pltpu.SemaphoreType.DMA,
        ],
    )


def kernel(hidden_states, gate_w, w1, w3, w2):
    b, s, d = hidden_states.shape
    x = hidden_states.reshape(b * s, d)
    logits, slot0, slot1, ss0, ss1, counts = _router_call(x, gate_w)
    slot0 = slot0.reshape(T)
    slot1 = slot1.reshape(T)
    xp = _dispatch()(x, slot0, slot1)
    yp = _experts_call(counts, xp, w1, w3, w2)
    out = _combine()(yp, ss0, ss1, slot0, slot1)
    return out.reshape(b, s, d), logits
